# Initial kernel scaffold; baseline (speedup 1.0000x reference)
import jax
import jax.numpy as jnp
from jax.experimental import pallas as pl


def kernel(x, edge_index, W1, b1, W2, b2, Wh1, bh1, Wh2, bh2):
    raise NotImplementedError("write your pallas kernel here")


# speedup vs baseline: 2.9649x; 2.9649x over previous
import functools

import jax
import jax.numpy as jnp
from jax import lax
from jax.experimental import pallas as pl
from jax.experimental.pallas import tpu as pltpu
from jax.experimental.pallas import tpu_sc as plsc

N = 10000
E = 320000
D = 128

NC = 2
NS = 16
NW = NC * NS

GROUP = 128
GROUPS_PER_W = 80
E_PAD = NW * GROUPS_PER_W * GROUP
AGG_ROWS = 10240
ZROWS_PER_TILE = AGG_ROWS // NS

_sc_mesh = plsc.VectorSubcoreMesh(core_axis_name="c", subcore_axis_name="s",
                                  num_cores=NC, num_subcores=NS)


@functools.partial(
    pl.kernel,
    out_type=jax.ShapeDtypeStruct((NC, AGG_ROWS, D), jnp.float32),
    mesh=_sc_mesh,
    scratch_types=[
        pltpu.VMEM((GROUPS_PER_W, GROUP), jnp.int32),
        pltpu.VMEM((GROUPS_PER_W, GROUP), jnp.int32),
        pltpu.VMEM((GROUP, D), jnp.float32),
        pltpu.VMEM_SHARED((AGG_ROWS, D), jnp.float32),
        pltpu.SemaphoreType.DMA,
    ],
)
def _sc_edge_agg(m_hbm, src_hbm, dst_hbm, out_hbm,
                 src_v, dst_v, rows_v, agg_sh, sem):
    c = lax.axis_index("c")
    s = lax.axis_index("s")
    w = c * NS + s

    def _zrow(i, _):
        for j in range(D // 16):
            rows_v[i, pl.ds(j * 16, 16)] = jnp.zeros((16,), jnp.float32)
        return _
    lax.fori_loop(0, GROUP, _zrow, None)
    for b in range(ZROWS_PER_TILE // GROUP):
        pltpu.sync_copy(rows_v, agg_sh.at[pl.ds(s * ZROWS_PER_TILE + b * GROUP,
                                                GROUP)])

    pltpu.sync_copy(src_hbm.at[pl.ds(w * GROUPS_PER_W, GROUPS_PER_W)], src_v)
    pltpu.sync_copy(dst_hbm.at[pl.ds(w * GROUPS_PER_W, GROUPS_PER_W)], dst_v)

    plsc.subcore_barrier()

    def _edge_group(g, _):
        pltpu.async_copy(m_hbm.at[src_v.at[g]], rows_v, sem).wait()
        pltpu.sync_copy(rows_v, agg_sh.at[dst_v.at[g]], add=True)
        return _
    lax.fori_loop(0, GROUPS_PER_W, _edge_group, None)

    plsc.subcore_barrier()

    pltpu.sync_copy(agg_sh.at[pl.ds(s * ZROWS_PER_TILE, ZROWS_PER_TILE)],
                    out_hbm.at[c, pl.ds(s * ZROWS_PER_TILE, ZROWS_PER_TILE)])


_BLK = 1000
_GRID = N // _BLK


def _mm_body(x_ref, w_ref, o_ref):
    o_ref[...] = jnp.dot(x_ref[...], w_ref[...],
                         preferred_element_type=jnp.float32)


def _tc_matmul(x, w):
    return pl.pallas_call(
        _mm_body,
        grid=(_GRID,),
        in_specs=[pl.BlockSpec((_BLK, D), lambda i: (i, 0)),
                  pl.BlockSpec((D, D), lambda i: (0, 0))],
        out_specs=pl.BlockSpec((_BLK, D), lambda i: (i, 0)),
        out_shape=jax.ShapeDtypeStruct((N, D), jnp.float32),
    )(x, w)


def _combine_mm_body(p0_ref, p1_ref, m_ref, b_ref, w_ref, o_ref):
    h = jax.nn.relu(p0_ref[0] + p1_ref[0] + m_ref[...] + b_ref[...])
    o_ref[...] = jnp.dot(h, w_ref[...], preferred_element_type=jnp.float32)


def _tc_combine_matmul(part, m, b, w):
    return pl.pallas_call(
        _combine_mm_body,
        grid=(_GRID,),
        in_specs=[pl.BlockSpec((1, _BLK, D), lambda i: (0, i, 0)),
                  pl.BlockSpec((1, _BLK, D), lambda i: (1, i, 0)),
                  pl.BlockSpec((_BLK, D), lambda i: (i, 0)),
                  pl.BlockSpec((1, D), lambda i: (0, 0)),
                  pl.BlockSpec((D, D), lambda i: (0, 0))],
        out_specs=pl.BlockSpec((_BLK, D), lambda i: (i, 0)),
        out_shape=jax.ShapeDtypeStruct((N, D), jnp.float32),
    )(part, part, m, b, w)


def _head_body(p0_ref, p1_ref, m_ref, b_ref, wh1_ref, bh1_ref, wh2_ref,
               bh2_ref, o_ref):
    h = jax.nn.relu(p0_ref[0] + p1_ref[0] + m_ref[...] + b_ref[...])
    h = jax.nn.relu(jnp.dot(h, wh1_ref[...],
                            preferred_element_type=jnp.float32) + bh1_ref[...])
    o_ref[...] = jnp.dot(h, wh2_ref[...],
                         preferred_element_type=jnp.float32) + bh2_ref[...]


def _tc_head(part, m, b, wh1, bh1, wh2, bh2):
    return pl.pallas_call(
        _head_body,
        grid=(_GRID,),
        in_specs=[pl.BlockSpec((1, _BLK, D), lambda i: (0, i, 0)),
                  pl.BlockSpec((1, _BLK, D), lambda i: (1, i, 0)),
                  pl.BlockSpec((_BLK, D), lambda i: (i, 0)),
                  pl.BlockSpec((1, D), lambda i: (0, 0)),
                  pl.BlockSpec((D, D), lambda i: (0, 0)),
                  pl.BlockSpec((1, D), lambda i: (0, 0)),
                  pl.BlockSpec((D, D), lambda i: (0, 0)),
                  pl.BlockSpec((1, D), lambda i: (0, 0))],
        out_specs=pl.BlockSpec((_BLK, D), lambda i: (i, 0)),
        out_shape=jax.ShapeDtypeStruct((N, D), jnp.float32),
    )(part, part, m, b, wh1, bh1, wh2, bh2)


def kernel(x, edge_index, W1, b1, W2, b2, Wh1, bh1, Wh2, bh2):
    src = edge_index[0]
    dst = edge_index[1]
    pad = E_PAD - E
    src_p = jnp.concatenate(
        [src, jnp.zeros((pad,), jnp.int32)]).reshape(E_PAD // GROUP, GROUP)
    dst_p = jnp.concatenate(
        [dst, jnp.full((pad,), N, jnp.int32)]).reshape(E_PAD // GROUP, GROUP)

    b1r = b1.reshape(1, D)
    b2r = b2.reshape(1, D)
    bh1r = bh1.reshape(1, D)
    bh2r = bh2.reshape(1, D)

    m1 = _tc_matmul(x, W1)
    p1 = _sc_edge_agg(m1, src_p, dst_p)
    m2 = _tc_combine_matmul(p1, m1, b1r, W2)
    p2 = _sc_edge_agg(m2, src_p, dst_p)
    return _tc_head(p2, m2, b2r, Wh1, bh1r, Wh2, bh2r)


# speedup vs baseline: 3.2807x; 1.1065x over previous
import functools

import jax
import jax.numpy as jnp
from jax import lax
from jax.experimental import pallas as pl
from jax.experimental.pallas import tpu as pltpu
from jax.experimental.pallas import tpu_sc as plsc

N = 10000
E = 320000
D = 128

NC = 2
NS = 16
NW = NC * NS

GROUP = 128
GROUPS_PER_W = 80
E_PAD = NW * GROUPS_PER_W * GROUP
AGG_ROWS = 10240
ZROWS_PER_TILE = AGG_ROWS // NS

_sc_mesh = plsc.VectorSubcoreMesh(core_axis_name="c", subcore_axis_name="s",
                                  num_cores=NC, num_subcores=NS)


IDXB = 8
NB = GROUPS_PER_W // IDXB


@functools.partial(
    pl.kernel,
    out_type=jax.ShapeDtypeStruct((NC, AGG_ROWS, D), jnp.float32),
    mesh=_sc_mesh,
    scratch_types=[
        pltpu.VMEM((IDXB, GROUP), jnp.int32),
        pltpu.VMEM((IDXB, GROUP), jnp.int32),
        pltpu.VMEM((GROUP, D), jnp.float32),
        pltpu.VMEM((GROUP, D), jnp.float32),
        pltpu.VMEM_SHARED((AGG_ROWS, D), jnp.float32),
        pltpu.SemaphoreType.DMA,
        pltpu.SemaphoreType.DMA,
        pltpu.SemaphoreType.DMA,
        pltpu.SemaphoreType.DMA,
    ],
)
def _sc_edge_agg(m_hbm, src_hbm, dst_hbm, out_hbm,
                 src_v, dst_v, rows_a, rows_b, agg_sh,
                 semg_a, semg_b, sems_a, sems_b):
    c = lax.axis_index("c")
    s = lax.axis_index("s")
    w = c * NS + s

    def _zrow(i, carry):
        for j in range(D // 16):
            rows_a[i, pl.ds(j * 16, 16)] = jnp.zeros((16,), jnp.float32)
        return carry
    lax.fori_loop(0, GROUP, _zrow, None)
    zdescs = [
        pltpu.async_copy(
            rows_a, agg_sh.at[pl.ds(s * ZROWS_PER_TILE + b * GROUP, GROUP)],
            semg_a)
        for b in range(ZROWS_PER_TILE // GROUP)
    ]
    for dsc in zdescs:
        dsc.wait()

    plsc.subcore_barrier()

    def _block(bi, carry):
        base = w * GROUPS_PER_W + bi * IDXB
        di0 = pltpu.async_copy(src_hbm.at[pl.ds(base, IDXB)], src_v, sems_a)
        di1 = pltpu.async_copy(dst_hbm.at[pl.ds(base, IDXB)], dst_v, sems_b)
        di0.wait()
        di1.wait()
        bufs = (rows_a, rows_b)
        gsems = (semg_a, semg_b)
        gd = [None, None]
        gd[0] = pltpu.async_copy(m_hbm.at[src_v.at[0]], bufs[0], gsems[0])
        for j in range(IDXB):
            p = j % 2
            if j + 1 < IDXB:
                gd[1 - p] = pltpu.async_copy(m_hbm.at[src_v.at[j + 1]],
                                             bufs[1 - p], gsems[1 - p])
            gd[p].wait()
            pltpu.sync_copy(bufs[p], agg_sh.at[dst_v.at[j]], add=True)
        return carry
    lax.fori_loop(0, NB, _block, None)

    plsc.subcore_barrier()

    pltpu.sync_copy(agg_sh.at[pl.ds(s * ZROWS_PER_TILE, ZROWS_PER_TILE)],
                    out_hbm.at[c, pl.ds(s * ZROWS_PER_TILE, ZROWS_PER_TILE)])


_BLK = 1000
_GRID = N // _BLK


def _mm_body(x_ref, w_ref, o_ref):
    o_ref[...] = jnp.dot(x_ref[...], w_ref[...],
                         preferred_element_type=jnp.float32)


def _tc_matmul(x, w):
    return pl.pallas_call(
        _mm_body,
        grid=(_GRID,),
        in_specs=[pl.BlockSpec((_BLK, D), lambda i: (i, 0)),
                  pl.BlockSpec((D, D), lambda i: (0, 0))],
        out_specs=pl.BlockSpec((_BLK, D), lambda i: (i, 0)),
        out_shape=jax.ShapeDtypeStruct((N, D), jnp.float32),
    )(x, w)


def _combine_mm_body(p0_ref, p1_ref, m_ref, b_ref, w_ref, o_ref):
    h = jax.nn.relu(p0_ref[0] + p1_ref[0] + m_ref[...] + b_ref[...])
    o_ref[...] = jnp.dot(h, w_ref[...], preferred_element_type=jnp.float32)


def _tc_combine_matmul(part, m, b, w):
    return pl.pallas_call(
        _combine_mm_body,
        grid=(_GRID,),
        in_specs=[pl.BlockSpec((1, _BLK, D), lambda i: (0, i, 0)),
                  pl.BlockSpec((1, _BLK, D), lambda i: (1, i, 0)),
                  pl.BlockSpec((_BLK, D), lambda i: (i, 0)),
                  pl.BlockSpec((1, D), lambda i: (0, 0)),
                  pl.BlockSpec((D, D), lambda i: (0, 0))],
        out_specs=pl.BlockSpec((_BLK, D), lambda i: (i, 0)),
        out_shape=jax.ShapeDtypeStruct((N, D), jnp.float32),
    )(part, part, m, b, w)


def _head_body(p0_ref, p1_ref, m_ref, b_ref, wh1_ref, bh1_ref, wh2_ref,
               bh2_ref, o_ref):
    h = jax.nn.relu(p0_ref[0] + p1_ref[0] + m_ref[...] + b_ref[...])
    h = jax.nn.relu(jnp.dot(h, wh1_ref[...],
                            preferred_element_type=jnp.float32) + bh1_ref[...])
    o_ref[...] = jnp.dot(h, wh2_ref[...],
                         preferred_element_type=jnp.float32) + bh2_ref[...]


def _tc_head(part, m, b, wh1, bh1, wh2, bh2):
    return pl.pallas_call(
        _head_body,
        grid=(_GRID,),
        in_specs=[pl.BlockSpec((1, _BLK, D), lambda i: (0, i, 0)),
                  pl.BlockSpec((1, _BLK, D), lambda i: (1, i, 0)),
                  pl.BlockSpec((_BLK, D), lambda i: (i, 0)),
                  pl.BlockSpec((1, D), lambda i: (0, 0)),
                  pl.BlockSpec((D, D), lambda i: (0, 0)),
                  pl.BlockSpec((1, D), lambda i: (0, 0)),
                  pl.BlockSpec((D, D), lambda i: (0, 0)),
                  pl.BlockSpec((1, D), lambda i: (0, 0))],
        out_specs=pl.BlockSpec((_BLK, D), lambda i: (i, 0)),
        out_shape=jax.ShapeDtypeStruct((N, D), jnp.float32),
    )(part, part, m, b, wh1, bh1, wh2, bh2)


def kernel(x, edge_index, W1, b1, W2, b2, Wh1, bh1, Wh2, bh2):
    src = edge_index[0]
    dst = edge_index[1]
    pad = E_PAD - E
    src_p = jnp.concatenate(
        [src, jnp.zeros((pad,), jnp.int32)]).reshape(E_PAD // GROUP, GROUP)
    dst_p = jnp.concatenate(
        [dst, jnp.full((pad,), N, jnp.int32)]).reshape(E_PAD // GROUP, GROUP)

    b1r = b1.reshape(1, D)
    b2r = b2.reshape(1, D)
    bh1r = bh1.reshape(1, D)
    bh2r = bh2.reshape(1, D)

    m1 = _tc_matmul(x, W1)
    p1 = _sc_edge_agg(m1, src_p, dst_p)
    m2 = _tc_combine_matmul(p1, m1, b1r, W2)
    p2 = _sc_edge_agg(m2, src_p, dst_p)
    return _tc_head(p2, m2, b2r, Wh1, bh1r, Wh2, bh2r)


# speedup vs baseline: 10.1328x; 3.0886x over previous
import functools

import jax
import jax.numpy as jnp
from jax import lax
from jax.experimental import pallas as pl
from jax.experimental.pallas import tpu as pltpu
from jax.experimental.pallas import tpu_sc as plsc

N = 10000
E = 320000
D = 128

NC = 2
NS = 16
NW = NC * NS

GROUP = 128
GROUPS_PER_W = 80
E_PAD = NW * GROUPS_PER_W * GROUP
AGG_ROWS = 10240
ZROWS_PER_TILE = AGG_ROWS // NS

_sc_mesh = plsc.VectorSubcoreMesh(core_axis_name="c", subcore_axis_name="s",
                                  num_cores=NC, num_subcores=NS)


IDXB = 8
NB = GROUPS_PER_W // IDXB


@functools.partial(
    pl.kernel,
    out_type=jax.ShapeDtypeStruct((NC, AGG_ROWS, D), jnp.float32),
    mesh=_sc_mesh,
    scratch_types=[
        pltpu.VMEM((IDXB, GROUP), jnp.int32),
        pltpu.VMEM((IDXB, GROUP), jnp.int32),
        pltpu.VMEM((GROUP, D), jnp.float32),
        pltpu.VMEM((GROUP, D), jnp.float32),
        pltpu.VMEM_SHARED((AGG_ROWS, D), jnp.float32),
        pltpu.SemaphoreType.DMA,
        pltpu.SemaphoreType.DMA,
        pltpu.SemaphoreType.DMA,
        pltpu.SemaphoreType.DMA,
    ],
)
def _sc_edge_agg(m_hbm, src_hbm, dst_hbm, out_hbm,
                 src_v, dst_v, rows_a, rows_b, agg_sh,
                 semg_a, semg_b, sems_a, sems_b):
    c = lax.axis_index("c")
    s = lax.axis_index("s")
    w = c * NS + s

    def _zrow(i, carry):
        for j in range(D // 16):
            rows_a[i, pl.ds(j * 16, 16)] = jnp.zeros((16,), jnp.float32)
        return carry
    lax.fori_loop(0, GROUP, _zrow, None)
    zdescs = [
        pltpu.async_copy(
            rows_a, agg_sh.at[pl.ds(s * ZROWS_PER_TILE + b * GROUP, GROUP)],
            semg_a)
        for b in range(ZROWS_PER_TILE // GROUP)
    ]
    for dsc in zdescs:
        dsc.wait()

    plsc.subcore_barrier()

    def _block(bi, carry):
        base = w * GROUPS_PER_W + bi * IDXB
        di0 = pltpu.async_copy(src_hbm.at[pl.ds(base, IDXB)], src_v, sems_a)
        di1 = pltpu.async_copy(dst_hbm.at[pl.ds(base, IDXB)], dst_v, sems_b)
        di0.wait()
        di1.wait()
        bufs = (rows_a, rows_b)
        gsems = (semg_a, semg_b)
        gd = [None, None]
        gd[0] = pltpu.async_copy(m_hbm.at[src_v.at[0]], bufs[0], gsems[0])
        for j in range(IDXB):
            p = j % 2
            if j + 1 < IDXB:
                gd[1 - p] = pltpu.async_copy(m_hbm.at[src_v.at[j + 1]],
                                             bufs[1 - p], gsems[1 - p])
            gd[p].wait()
            pltpu.sync_copy(bufs[p], agg_sh.at[dst_v.at[j]], add=True)
        return carry
    lax.fori_loop(0, NB, _block, None)

    plsc.subcore_barrier()

    pltpu.sync_copy(agg_sh.at[pl.ds(s * ZROWS_PER_TILE, ZROWS_PER_TILE)],
                    out_hbm.at[c, pl.ds(s * ZROWS_PER_TILE, ZROWS_PER_TILE)])


_BLK = 1000
_GRID = N // _BLK


def _mm_body(x_ref, w_ref, o_ref):
    o_ref[...] = jnp.dot(x_ref[...], w_ref[...],
                         preferred_element_type=jnp.float32)


def _tc_matmul(x, w):
    return pl.pallas_call(
        _mm_body,
        grid=(_GRID,),
        in_specs=[pl.BlockSpec((_BLK, D), lambda i: (i, 0)),
                  pl.BlockSpec((D, D), lambda i: (0, 0))],
        out_specs=pl.BlockSpec((_BLK, D), lambda i: (i, 0)),
        out_shape=jax.ShapeDtypeStruct((N, D), jnp.float32),
    )(x, w)


def _combine_mm_body(p0_ref, p1_ref, m_ref, b_ref, w_ref, o_ref):
    h = jax.nn.relu(p0_ref[0] + p1_ref[0] + m_ref[...] + b_ref[...])
    o_ref[...] = jnp.dot(h, w_ref[...], preferred_element_type=jnp.float32)


def _tc_combine_matmul(part, m, b, w):
    return pl.pallas_call(
        _combine_mm_body,
        grid=(_GRID,),
        in_specs=[pl.BlockSpec((1, _BLK, D), lambda i: (0, i, 0)),
                  pl.BlockSpec((1, _BLK, D), lambda i: (1, i, 0)),
                  pl.BlockSpec((_BLK, D), lambda i: (i, 0)),
                  pl.BlockSpec((1, D), lambda i: (0, 0)),
                  pl.BlockSpec((D, D), lambda i: (0, 0))],
        out_specs=pl.BlockSpec((_BLK, D), lambda i: (i, 0)),
        out_shape=jax.ShapeDtypeStruct((N, D), jnp.float32),
    )(part, part, m, b, w)


def _head_body(p0_ref, p1_ref, m_ref, b_ref, wh1_ref, bh1_ref, wh2_ref,
               bh2_ref, o_ref):
    h = jax.nn.relu(p0_ref[0] + p1_ref[0] + m_ref[...] + b_ref[...])
    h = jax.nn.relu(jnp.dot(h, wh1_ref[...],
                            preferred_element_type=jnp.float32) + bh1_ref[...])
    o_ref[...] = jnp.dot(h, wh2_ref[...],
                         preferred_element_type=jnp.float32) + bh2_ref[...]


def _tc_head(part, m, b, wh1, bh1, wh2, bh2):
    return pl.pallas_call(
        _head_body,
        grid=(_GRID,),
        in_specs=[pl.BlockSpec((1, _BLK, D), lambda i: (0, i, 0)),
                  pl.BlockSpec((1, _BLK, D), lambda i: (1, i, 0)),
                  pl.BlockSpec((_BLK, D), lambda i: (i, 0)),
                  pl.BlockSpec((1, D), lambda i: (0, 0)),
                  pl.BlockSpec((D, D), lambda i: (0, 0)),
                  pl.BlockSpec((1, D), lambda i: (0, 0)),
                  pl.BlockSpec((D, D), lambda i: (0, 0)),
                  pl.BlockSpec((1, D), lambda i: (0, 0))],
        out_specs=pl.BlockSpec((_BLK, D), lambda i: (i, 0)),
        out_shape=jax.ShapeDtypeStruct((N, D), jnp.float32),
    )(part, part, m, b, wh1, bh1, wh2, bh2)


def kernel(x, edge_index, W1, b1, W2, b2, Wh1, bh1, Wh2, bh2):
    src = edge_index[0]
    dst = edge_index[1]
    pad = E_PAD - E
    pad_i = jnp.arange(pad, dtype=jnp.int32)
    src_p = jnp.concatenate(
        [src, pad_i % N]).reshape(E_PAD // GROUP, GROUP)
    dst_p = jnp.concatenate(
        [dst, N + pad_i % (AGG_ROWS - N)]).reshape(E_PAD // GROUP, GROUP)

    b1r = b1.reshape(1, D)
    b2r = b2.reshape(1, D)
    bh1r = bh1.reshape(1, D)
    bh2r = bh2.reshape(1, D)

    m1 = _tc_matmul(x, W1)
    p1 = _sc_edge_agg(m1, src_p, dst_p)
    m2 = _tc_combine_matmul(p1, m1, b1r, W2)
    p2 = _sc_edge_agg(m2, src_p, dst_p)
    return _tc_head(p2, m2, b2r, Wh1, bh1r, Wh2, bh2r)


# speedup vs baseline: 11.4632x; 1.1313x over previous
import functools

import jax
import jax.numpy as jnp
from jax import lax
from jax.experimental import pallas as pl
from jax.experimental.pallas import tpu as pltpu
from jax.experimental.pallas import tpu_sc as plsc

N = 10000
E = 320000
D = 128

NC = 2
NS = 16
NW = NC * NS

GROUP = 128
GROUPS_PER_W = 80
E_PAD = NW * GROUPS_PER_W * GROUP
AGG_ROWS = 10240
ZROWS_PER_TILE = AGG_ROWS // NS

_sc_mesh = plsc.VectorSubcoreMesh(core_axis_name="c", subcore_axis_name="s",
                                  num_cores=NC, num_subcores=NS)


IDXB = 8
NB = GROUPS_PER_W // IDXB


@functools.partial(
    pl.kernel,
    out_type=jax.ShapeDtypeStruct((NC, AGG_ROWS, D), jnp.float32),
    mesh=_sc_mesh,
    scratch_types=[
        pltpu.VMEM((IDXB, GROUP), jnp.int32),
        pltpu.VMEM((IDXB, GROUP), jnp.int32),
        pltpu.VMEM((IDXB, GROUP), jnp.int32),
        pltpu.VMEM((IDXB, GROUP), jnp.int32),
        pltpu.VMEM((GROUP, D), jnp.float32),
        pltpu.VMEM((GROUP, D), jnp.float32),
        pltpu.VMEM_SHARED((AGG_ROWS, D), jnp.float32),
        pltpu.SemaphoreType.DMA,
        pltpu.SemaphoreType.DMA,
        pltpu.SemaphoreType.DMA,
        pltpu.SemaphoreType.DMA,
        pltpu.SemaphoreType.DMA,
    ],
)
def _sc_edge_agg(m_hbm, src_hbm, dst_hbm, out_hbm,
                 isrc_a, idst_a, isrc_b, idst_b, rows_0, rows_1, agg_sh,
                 semg_0, semg_1, sems, semi_a, semi_b):
    c = lax.axis_index("c")
    s = lax.axis_index("s")
    w = c * NS + s

    def _zrow(i, carry):
        for j in range(D // 16):
            rows_0[i, pl.ds(j * 16, 16)] = jnp.zeros((16,), jnp.float32)
        return carry
    lax.fori_loop(0, GROUP, _zrow, None)
    zdescs = [
        pltpu.async_copy(
            rows_0, agg_sh.at[pl.ds(s * ZROWS_PER_TILE + b * GROUP, GROUP)],
            semg_0)
        for b in range(ZROWS_PER_TILE // GROUP)
    ]
    for dsc in zdescs:
        dsc.wait()

    base0 = w * GROUPS_PER_W
    d0 = pltpu.async_copy(src_hbm.at[pl.ds(base0, IDXB)], isrc_a, semi_a)
    d1 = pltpu.async_copy(dst_hbm.at[pl.ds(base0, IDXB)], idst_a, semi_a)
    pltpu.async_copy(src_hbm.at[pl.ds(base0 + IDXB, IDXB)], isrc_b, semi_b)
    pltpu.async_copy(dst_hbm.at[pl.ds(base0 + IDXB, IDXB)], idst_b, semi_b)
    d0.wait()
    d1.wait()

    plsc.subcore_barrier()

    pltpu.async_copy(m_hbm.at[isrc_a.at[0]], rows_0, semg_0)

    rows = (rows_0, rows_1)
    gsems = (semg_0, semg_1)

    def _wait_gather(p):
        pltpu.make_async_copy(m_hbm.at[pl.ds(0, GROUP)], rows[p],
                              gsems[p]).wait()

    def _wait_idx(sem, isrc, idst):
        pltpu.make_async_copy(src_hbm.at[pl.ds(0, IDXB)], isrc, sem).wait()
        pltpu.make_async_copy(dst_hbm.at[pl.ds(0, IDXB)], idst, sem).wait()

    def _block(i, half, isrc, idst, nsrc, ndst, isem, nsem):
        bi = i * 2 + half
        gd = [None, None]
        for j in range(IDXB):
            p = j % 2
            if j + 1 < IDXB:
                gd[1 - p] = pltpu.async_copy(m_hbm.at[isrc.at[j + 1]],
                                             rows[1 - p], gsems[1 - p])
            else:
                def _next_gather():
                    _wait_idx(nsem, nsrc, ndst)
                    pltpu.async_copy(m_hbm.at[nsrc.at[0]], rows[1 - p],
                                     gsems[1 - p])
                if half == 0:
                    _next_gather()
                else:
                    pl.when(i < NB // 2 - 1)(_next_gather)
            if j == 0:
                _wait_gather(0)
            else:
                gd[p].wait()
            pltpu.sync_copy(rows[p], agg_sh.at[idst.at[j]], add=True)

        def _prefetch():
            nbase = w * GROUPS_PER_W + (bi + 2) * IDXB
            pltpu.async_copy(src_hbm.at[pl.ds(nbase, IDXB)], isrc, isem)
            pltpu.async_copy(dst_hbm.at[pl.ds(nbase, IDXB)], idst, isem)
        pl.when(i < NB // 2 - 1)(_prefetch)

    def _pipe(i, carry):
        _block(i, 0, isrc_a, idst_a, isrc_b, idst_b, semi_a, semi_b)
        _block(i, 1, isrc_b, idst_b, isrc_a, idst_a, semi_b, semi_a)
        return carry
    lax.fori_loop(0, NB // 2, _pipe, None)

    plsc.subcore_barrier()

    pltpu.sync_copy(agg_sh.at[pl.ds(s * ZROWS_PER_TILE, ZROWS_PER_TILE)],
                    out_hbm.at[c, pl.ds(s * ZROWS_PER_TILE, ZROWS_PER_TILE)])


_BLK = 1000
_GRID = N // _BLK


def _mm_body(x_ref, w_ref, o_ref):
    o_ref[...] = jnp.dot(x_ref[...], w_ref[...],
                         preferred_element_type=jnp.float32)


def _tc_matmul(x, w):
    return pl.pallas_call(
        _mm_body,
        grid=(_GRID,),
        in_specs=[pl.BlockSpec((_BLK, D), lambda i: (i, 0)),
                  pl.BlockSpec((D, D), lambda i: (0, 0))],
        out_specs=pl.BlockSpec((_BLK, D), lambda i: (i, 0)),
        out_shape=jax.ShapeDtypeStruct((N, D), jnp.float32),
    )(x, w)


def _combine_mm_body(p0_ref, p1_ref, m_ref, b_ref, w_ref, o_ref):
    h = jax.nn.relu(p0_ref[0] + p1_ref[0] + m_ref[...] + b_ref[...])
    o_ref[...] = jnp.dot(h, w_ref[...], preferred_element_type=jnp.float32)


def _tc_combine_matmul(part, m, b, w):
    return pl.pallas_call(
        _combine_mm_body,
        grid=(_GRID,),
        in_specs=[pl.BlockSpec((1, _BLK, D), lambda i: (0, i, 0)),
                  pl.BlockSpec((1, _BLK, D), lambda i: (1, i, 0)),
                  pl.BlockSpec((_BLK, D), lambda i: (i, 0)),
                  pl.BlockSpec((1, D), lambda i: (0, 0)),
                  pl.BlockSpec((D, D), lambda i: (0, 0))],
        out_specs=pl.BlockSpec((_BLK, D), lambda i: (i, 0)),
        out_shape=jax.ShapeDtypeStruct((N, D), jnp.float32),
    )(part, part, m, b, w)


def _head_body(p0_ref, p1_ref, m_ref, b_ref, wh1_ref, bh1_ref, wh2_ref,
               bh2_ref, o_ref):
    h = jax.nn.relu(p0_ref[0] + p1_ref[0] + m_ref[...] + b_ref[...])
    h = jax.nn.relu(jnp.dot(h, wh1_ref[...],
                            preferred_element_type=jnp.float32) + bh1_ref[...])
    o_ref[...] = jnp.dot(h, wh2_ref[...],
                         preferred_element_type=jnp.float32) + bh2_ref[...]


def _tc_head(part, m, b, wh1, bh1, wh2, bh2):
    return pl.pallas_call(
        _head_body,
        grid=(_GRID,),
        in_specs=[pl.BlockSpec((1, _BLK, D), lambda i: (0, i, 0)),
                  pl.BlockSpec((1, _BLK, D), lambda i: (1, i, 0)),
                  pl.BlockSpec((_BLK, D), lambda i: (i, 0)),
                  pl.BlockSpec((1, D), lambda i: (0, 0)),
                  pl.BlockSpec((D, D), lambda i: (0, 0)),
                  pl.BlockSpec((1, D), lambda i: (0, 0)),
                  pl.BlockSpec((D, D), lambda i: (0, 0)),
                  pl.BlockSpec((1, D), lambda i: (0, 0))],
        out_specs=pl.BlockSpec((_BLK, D), lambda i: (i, 0)),
        out_shape=jax.ShapeDtypeStruct((N, D), jnp.float32),
    )(part, part, m, b, wh1, bh1, wh2, bh2)


def kernel(x, edge_index, W1, b1, W2, b2, Wh1, bh1, Wh2, bh2):
    src = edge_index[0]
    dst = edge_index[1]
    pad = E_PAD - E
    pad_i = jnp.arange(pad, dtype=jnp.int32)
    src_p = jnp.concatenate(
        [src, pad_i % N]).reshape(E_PAD // GROUP, GROUP)
    dst_p = jnp.concatenate(
        [dst, N + pad_i % (AGG_ROWS - N)]).reshape(E_PAD // GROUP, GROUP)

    b1r = b1.reshape(1, D)
    b2r = b2.reshape(1, D)
    bh1r = bh1.reshape(1, D)
    bh2r = bh2.reshape(1, D)

    m1 = _tc_matmul(x, W1)
    p1 = _sc_edge_agg(m1, src_p, dst_p)
    m2 = _tc_combine_matmul(p1, m1, b1r, W2)
    p2 = _sc_edge_agg(m2, src_p, dst_p)
    return _tc_head(p2, m2, b2r, Wh1, bh1r, Wh2, bh2r)


# speedup vs baseline: 11.4722x; 1.0008x over previous
import functools

import jax
import jax.numpy as jnp
from jax import lax
from jax.experimental import pallas as pl
from jax.experimental.pallas import tpu as pltpu
from jax.experimental.pallas import tpu_sc as plsc

N = 10000
E = 320000
D = 128

NC = 2
NS = 16
NW = NC * NS

GROUP = 128
GROUPS_PER_W = 80
E_PAD = NW * GROUPS_PER_W * GROUP
AGG_ROWS = 10240
ZROWS_PER_TILE = AGG_ROWS // NS

_sc_mesh = plsc.VectorSubcoreMesh(core_axis_name="c", subcore_axis_name="s",
                                  num_cores=NC, num_subcores=NS)


IDXB = 8
NB = GROUPS_PER_W // IDXB


@functools.partial(
    pl.kernel,
    out_type=jax.ShapeDtypeStruct((NC, AGG_ROWS, D), jnp.float32),
    mesh=_sc_mesh,
    scratch_types=[
        pltpu.VMEM((IDXB, GROUP), jnp.int32),
        pltpu.VMEM((IDXB, GROUP), jnp.int32),
        pltpu.VMEM((IDXB, GROUP), jnp.int32),
        pltpu.VMEM((IDXB, GROUP), jnp.int32),
        pltpu.VMEM((GROUP, D), jnp.float32),
        pltpu.VMEM((GROUP, D), jnp.float32),
        pltpu.VMEM_SHARED((AGG_ROWS, D), jnp.float32),
        pltpu.SemaphoreType.DMA,
        pltpu.SemaphoreType.DMA,
        pltpu.SemaphoreType.DMA,
        pltpu.SemaphoreType.DMA,
        pltpu.SemaphoreType.DMA,
        pltpu.SemaphoreType.DMA,
    ],
)
def _sc_edge_agg(m_hbm, src_hbm, dst_hbm, out_hbm,
                 isrc_a, idst_a, isrc_b, idst_b, rows_0, rows_1, agg_sh,
                 semg_0, semg_1, sems_0, sems_1, semi_a, semi_b):
    c = lax.axis_index("c")
    s = lax.axis_index("s")
    w = c * NS + s

    def _zrow(i, carry):
        for j in range(D // 16):
            rows_0[i, pl.ds(j * 16, 16)] = jnp.zeros((16,), jnp.float32)
        return carry
    lax.fori_loop(0, GROUP, _zrow, None)
    zdescs = [
        pltpu.async_copy(
            rows_0, agg_sh.at[pl.ds(s * ZROWS_PER_TILE + b * GROUP, GROUP)],
            semg_0)
        for b in range(ZROWS_PER_TILE // GROUP)
    ]
    for dsc in zdescs:
        dsc.wait()

    base0 = w * GROUPS_PER_W
    d0 = pltpu.async_copy(src_hbm.at[pl.ds(base0, IDXB)], isrc_a, semi_a)
    d1 = pltpu.async_copy(dst_hbm.at[pl.ds(base0, IDXB)], idst_a, semi_a)
    pltpu.async_copy(src_hbm.at[pl.ds(base0 + IDXB, IDXB)], isrc_b, semi_b)
    pltpu.async_copy(dst_hbm.at[pl.ds(base0 + IDXB, IDXB)], idst_b, semi_b)
    d0.wait()
    d1.wait()

    plsc.subcore_barrier()

    pltpu.async_copy(m_hbm.at[isrc_a.at[0]], rows_0, semg_0)

    rows = (rows_0, rows_1)
    gsems = (semg_0, semg_1)
    ssems = (sems_0, sems_1)

    def _wait_gather(p):
        pltpu.make_async_copy(m_hbm.at[pl.ds(0, GROUP)], rows[p],
                              gsems[p]).wait()

    def _wait_scatter(p):
        pltpu.make_async_copy(m_hbm.at[pl.ds(0, GROUP)], rows[p],
                              ssems[p]).wait()

    def _wait_idx(sem, isrc, idst):
        pltpu.make_async_copy(src_hbm.at[pl.ds(0, IDXB)], isrc, sem).wait()
        pltpu.make_async_copy(dst_hbm.at[pl.ds(0, IDXB)], idst, sem).wait()

    def _block(i, half, isrc, idst, nsrc, ndst, isem, nsem):
        bi = i * 2 + half
        gd = [None, None]
        sd = [None, None]
        for j in range(IDXB):
            p = j % 2
            if j == 0:
                if half == 0:
                    pl.when(i > 0)(lambda: _wait_scatter(1))
                else:
                    _wait_scatter(1)
            else:
                sd[1 - p].wait()
            if j + 1 < IDXB:
                gd[1 - p] = pltpu.async_copy(m_hbm.at[isrc.at[j + 1]],
                                             rows[1 - p], gsems[1 - p])
            else:
                def _next_gather():
                    _wait_idx(nsem, nsrc, ndst)
                    pltpu.async_copy(m_hbm.at[nsrc.at[0]], rows[1 - p],
                                     gsems[1 - p])
                if half == 0:
                    _next_gather()
                else:
                    pl.when(i < NB // 2 - 1)(_next_gather)
            if j == 0:
                _wait_gather(0)
            else:
                gd[p].wait()
            sd[p] = pltpu.async_copy(rows[p], agg_sh.at[idst.at[j]],
                                     ssems[p], add=True)

        def _prefetch():
            nbase = w * GROUPS_PER_W + (bi + 2) * IDXB
            pltpu.async_copy(src_hbm.at[pl.ds(nbase, IDXB)], isrc, isem)
            pltpu.async_copy(dst_hbm.at[pl.ds(nbase, IDXB)], idst, isem)
        pl.when(i < NB // 2 - 1)(_prefetch)

    def _pipe(i, carry):
        _block(i, 0, isrc_a, idst_a, isrc_b, idst_b, semi_a, semi_b)
        _block(i, 1, isrc_b, idst_b, isrc_a, idst_a, semi_b, semi_a)
        return carry
    lax.fori_loop(0, NB // 2, _pipe, None)
    _wait_scatter(1)

    plsc.subcore_barrier()

    pltpu.sync_copy(agg_sh.at[pl.ds(s * ZROWS_PER_TILE, ZROWS_PER_TILE)],
                    out_hbm.at[c, pl.ds(s * ZROWS_PER_TILE, ZROWS_PER_TILE)])


_BLK = 1000
_GRID = N // _BLK


def _mm_body(x_ref, w_ref, o_ref):
    o_ref[...] = jnp.dot(x_ref[...], w_ref[...],
                         preferred_element_type=jnp.float32)


def _tc_matmul(x, w):
    return pl.pallas_call(
        _mm_body,
        grid=(_GRID,),
        in_specs=[pl.BlockSpec((_BLK, D), lambda i: (i, 0)),
                  pl.BlockSpec((D, D), lambda i: (0, 0))],
        out_specs=pl.BlockSpec((_BLK, D), lambda i: (i, 0)),
        out_shape=jax.ShapeDtypeStruct((N, D), jnp.float32),
    )(x, w)


def _combine_mm_body(p0_ref, p1_ref, m_ref, b_ref, w_ref, o_ref):
    h = jax.nn.relu(p0_ref[0] + p1_ref[0] + m_ref[...] + b_ref[...])
    o_ref[...] = jnp.dot(h, w_ref[...], preferred_element_type=jnp.float32)


def _tc_combine_matmul(part, m, b, w):
    return pl.pallas_call(
        _combine_mm_body,
        grid=(_GRID,),
        in_specs=[pl.BlockSpec((1, _BLK, D), lambda i: (0, i, 0)),
                  pl.BlockSpec((1, _BLK, D), lambda i: (1, i, 0)),
                  pl.BlockSpec((_BLK, D), lambda i: (i, 0)),
                  pl.BlockSpec((1, D), lambda i: (0, 0)),
                  pl.BlockSpec((D, D), lambda i: (0, 0))],
        out_specs=pl.BlockSpec((_BLK, D), lambda i: (i, 0)),
        out_shape=jax.ShapeDtypeStruct((N, D), jnp.float32),
    )(part, part, m, b, w)


def _head_body(p0_ref, p1_ref, m_ref, b_ref, wh1_ref, bh1_ref, wh2_ref,
               bh2_ref, o_ref):
    h = jax.nn.relu(p0_ref[0] + p1_ref[0] + m_ref[...] + b_ref[...])
    h = jax.nn.relu(jnp.dot(h, wh1_ref[...],
                            preferred_element_type=jnp.float32) + bh1_ref[...])
    o_ref[...] = jnp.dot(h, wh2_ref[...],
                         preferred_element_type=jnp.float32) + bh2_ref[...]


def _tc_head(part, m, b, wh1, bh1, wh2, bh2):
    return pl.pallas_call(
        _head_body,
        grid=(_GRID,),
        in_specs=[pl.BlockSpec((1, _BLK, D), lambda i: (0, i, 0)),
                  pl.BlockSpec((1, _BLK, D), lambda i: (1, i, 0)),
                  pl.BlockSpec((_BLK, D), lambda i: (i, 0)),
                  pl.BlockSpec((1, D), lambda i: (0, 0)),
                  pl.BlockSpec((D, D), lambda i: (0, 0)),
                  pl.BlockSpec((1, D), lambda i: (0, 0)),
                  pl.BlockSpec((D, D), lambda i: (0, 0)),
                  pl.BlockSpec((1, D), lambda i: (0, 0))],
        out_specs=pl.BlockSpec((_BLK, D), lambda i: (i, 0)),
        out_shape=jax.ShapeDtypeStruct((N, D), jnp.float32),
    )(part, part, m, b, wh1, bh1, wh2, bh2)


def kernel(x, edge_index, W1, b1, W2, b2, Wh1, bh1, Wh2, bh2):
    src = edge_index[0]
    dst = edge_index[1]
    pad = E_PAD - E
    pad_i = jnp.arange(pad, dtype=jnp.int32)
    src_p = jnp.concatenate(
        [src, pad_i % N]).reshape(E_PAD // GROUP, GROUP)
    dst_p = jnp.concatenate(
        [dst, N + pad_i % (AGG_ROWS - N)]).reshape(E_PAD // GROUP, GROUP)

    b1r = b1.reshape(1, D)
    b2r = b2.reshape(1, D)
    bh1r = bh1.reshape(1, D)
    bh2r = bh2.reshape(1, D)

    m1 = _tc_matmul(x, W1)
    p1 = _sc_edge_agg(m1, src_p, dst_p)
    m2 = _tc_combine_matmul(p1, m1, b1r, W2)
    p2 = _sc_edge_agg(m2, src_p, dst_p)
    return _tc_head(p2, m2, b2r, Wh1, bh1r, Wh2, bh2r)


# speedup vs baseline: 11.5534x; 1.0071x over previous
import functools

import jax
import jax.numpy as jnp
from jax import lax
from jax.experimental import pallas as pl
from jax.experimental.pallas import tpu as pltpu
from jax.experimental.pallas import tpu_sc as plsc

N = 10000
E = 320000
D = 128

NC = 2
NS = 16
NW = NC * NS

GROUP = 128
GROUPS_PER_W = 80
E_PAD = NW * GROUPS_PER_W * GROUP
AGG_ROWS = 10240
ZROWS_PER_TILE = AGG_ROWS // NS

_sc_mesh = plsc.VectorSubcoreMesh(core_axis_name="c", subcore_axis_name="s",
                                  num_cores=NC, num_subcores=NS)


IDXB = 8
NB = GROUPS_PER_W // IDXB


@functools.partial(
    pl.kernel,
    out_type=jax.ShapeDtypeStruct((NC, AGG_ROWS, D), jnp.float32),
    mesh=_sc_mesh,
    scratch_types=[
        pltpu.VMEM((IDXB, GROUP), jnp.int32),
        pltpu.VMEM((IDXB, GROUP), jnp.int32),
        pltpu.VMEM((IDXB, GROUP), jnp.int32),
        pltpu.VMEM((IDXB, GROUP), jnp.int32),
        pltpu.VMEM((GROUP, D), jnp.float32),
        pltpu.VMEM((GROUP, D), jnp.float32),
        pltpu.VMEM_SHARED((AGG_ROWS, D), jnp.float32),
        pltpu.SemaphoreType.DMA,
        pltpu.SemaphoreType.DMA,
        pltpu.SemaphoreType.DMA,
        pltpu.SemaphoreType.DMA,
        pltpu.SemaphoreType.DMA,
        pltpu.SemaphoreType.DMA,
    ],
)
def _sc_edge_agg(m_hbm, src_hbm, dst_hbm, out_hbm,
                 isrc_a, idst_a, isrc_b, idst_b, rows_0, rows_1, agg_sh,
                 semg_0, semg_1, sems_0, sems_1, semi_a, semi_b):
    c = lax.axis_index("c")
    s = lax.axis_index("s")
    w = c * NS + s

    def _zrow(i, carry):
        for j in range(D // 16):
            rows_0[i, pl.ds(j * 16, 16)] = jnp.zeros((16,), jnp.float32)
        return carry
    lax.fori_loop(0, GROUP, _zrow, None)
    zdescs = [
        pltpu.async_copy(
            rows_0, agg_sh.at[pl.ds(s * ZROWS_PER_TILE + b * GROUP, GROUP)],
            semg_0)
        for b in range(ZROWS_PER_TILE // GROUP)
    ]
    for dsc in zdescs:
        dsc.wait()

    base0 = w * GROUPS_PER_W
    d0 = pltpu.async_copy(src_hbm.at[pl.ds(base0, IDXB)], isrc_a, semi_a)
    d1 = pltpu.async_copy(dst_hbm.at[pl.ds(base0, IDXB)], idst_a, semi_a)
    pltpu.async_copy(src_hbm.at[pl.ds(base0 + IDXB, IDXB)], isrc_b, semi_b)
    pltpu.async_copy(dst_hbm.at[pl.ds(base0 + IDXB, IDXB)], idst_b, semi_b)
    d0.wait()
    d1.wait()

    plsc.subcore_barrier()

    pltpu.async_copy(m_hbm.at[isrc_a.at[0]], rows_0, semg_0)

    rows = (rows_0, rows_1)
    gsems = (semg_0, semg_1)
    ssems = (sems_0, sems_1)

    def _wait_gather(p):
        pltpu.make_async_copy(m_hbm.at[pl.ds(0, GROUP)], rows[p],
                              gsems[p]).wait()

    def _wait_scatter(p):
        pltpu.make_async_copy(m_hbm.at[pl.ds(0, GROUP)], rows[p],
                              ssems[p]).wait()

    def _wait_idx(sem, isrc, idst):
        pltpu.make_async_copy(src_hbm.at[pl.ds(0, IDXB)], isrc, sem).wait()
        pltpu.make_async_copy(dst_hbm.at[pl.ds(0, IDXB)], idst, sem).wait()

    def _block(i, half, isrc, idst, nsrc, ndst, isem, nsem):
        bi = i * 2 + half
        gd = [None, None]
        sd = [None, None]
        for j in range(IDXB):
            p = j % 2
            if j == 0:
                if half == 0:
                    pl.when(i > 0)(lambda: _wait_scatter(1))
                else:
                    _wait_scatter(1)
            else:
                sd[1 - p].wait()
            if j + 1 < IDXB:
                gd[1 - p] = pltpu.async_copy(m_hbm.at[isrc.at[j + 1]],
                                             rows[1 - p], gsems[1 - p])
            else:
                def _next_gather():
                    _wait_idx(nsem, nsrc, ndst)
                    pltpu.async_copy(m_hbm.at[nsrc.at[0]], rows[1 - p],
                                     gsems[1 - p])
                if half == 0:
                    _next_gather()
                else:
                    pl.when(i < NB // 2 - 1)(_next_gather)
            if j == 0:
                _wait_gather(0)
            else:
                gd[p].wait()
            sd[p] = pltpu.async_copy(rows[p], agg_sh.at[idst.at[j]],
                                     ssems[p], add=True)

        def _prefetch():
            nbase = w * GROUPS_PER_W + (bi + 2) * IDXB
            pltpu.async_copy(src_hbm.at[pl.ds(nbase, IDXB)], isrc, isem)
            pltpu.async_copy(dst_hbm.at[pl.ds(nbase, IDXB)], idst, isem)
        pl.when(i < NB // 2 - 1)(_prefetch)

    def _pipe(i, carry):
        _block(i, 0, isrc_a, idst_a, isrc_b, idst_b, semi_a, semi_b)
        _block(i, 1, isrc_b, idst_b, isrc_a, idst_a, semi_b, semi_a)
        return carry
    lax.fori_loop(0, NB // 2, _pipe, None)
    _wait_scatter(1)

    plsc.subcore_barrier()

    pltpu.sync_copy(agg_sh.at[pl.ds(s * ZROWS_PER_TILE, ZROWS_PER_TILE)],
                    out_hbm.at[c, pl.ds(s * ZROWS_PER_TILE, ZROWS_PER_TILE)])


_BLK = 1000
_GRID = N // _BLK

_NGRP = E_PAD // GROUP
_PBLK = _NGRP // 10


def _prep_body(es_ref, ed_ref, os_ref, od_ref):
    i = pl.program_id(0)
    r = lax.broadcasted_iota(jnp.int32, (_PBLK, D), 0)
    cidx = lax.broadcasted_iota(jnp.int32, (_PBLK, D), 1)
    elem = (i * _PBLK + r) * GROUP + cidx
    pe = elem - E
    os_ref[...] = jnp.where(elem < E, es_ref[0], pe % N)
    od_ref[...] = jnp.where(elem < E, ed_ref[0], N + pe % (AGG_ROWS - N))


def _tc_prep(eidx3):
    return pl.pallas_call(
        _prep_body,
        grid=(10,),
        in_specs=[pl.BlockSpec((1, _PBLK, D), lambda i: (0, i, 0)),
                  pl.BlockSpec((1, _PBLK, D), lambda i: (1, i, 0))],
        out_specs=[pl.BlockSpec((_PBLK, D), lambda i: (i, 0)),
                   pl.BlockSpec((_PBLK, D), lambda i: (i, 0))],
        out_shape=[jax.ShapeDtypeStruct((_NGRP, D), jnp.int32),
                   jax.ShapeDtypeStruct((_NGRP, D), jnp.int32)],
    )(eidx3, eidx3)


def _mm_body(x_ref, w_ref, o_ref):
    o_ref[...] = jnp.dot(x_ref[...], w_ref[...],
                         preferred_element_type=jnp.float32)


def _tc_matmul(x, w):
    return pl.pallas_call(
        _mm_body,
        grid=(_GRID,),
        in_specs=[pl.BlockSpec((_BLK, D), lambda i: (i, 0)),
                  pl.BlockSpec((D, D), lambda i: (0, 0))],
        out_specs=pl.BlockSpec((_BLK, D), lambda i: (i, 0)),
        out_shape=jax.ShapeDtypeStruct((N, D), jnp.float32),
    )(x, w)


def _combine_mm_body(p0_ref, p1_ref, m_ref, b_ref, w_ref, o_ref):
    h = jax.nn.relu(p0_ref[0] + p1_ref[0] + m_ref[...] + b_ref[...])
    o_ref[...] = jnp.dot(h, w_ref[...], preferred_element_type=jnp.float32)


def _tc_combine_matmul(part, m, b, w):
    return pl.pallas_call(
        _combine_mm_body,
        grid=(_GRID,),
        in_specs=[pl.BlockSpec((1, _BLK, D), lambda i: (0, i, 0)),
                  pl.BlockSpec((1, _BLK, D), lambda i: (1, i, 0)),
                  pl.BlockSpec((_BLK, D), lambda i: (i, 0)),
                  pl.BlockSpec((1, D), lambda i: (0, 0)),
                  pl.BlockSpec((D, D), lambda i: (0, 0))],
        out_specs=pl.BlockSpec((_BLK, D), lambda i: (i, 0)),
        out_shape=jax.ShapeDtypeStruct((N, D), jnp.float32),
    )(part, part, m, b, w)


def _head_body(p0_ref, p1_ref, m_ref, b_ref, wh1_ref, bh1_ref, wh2_ref,
               bh2_ref, o_ref):
    h = jax.nn.relu(p0_ref[0] + p1_ref[0] + m_ref[...] + b_ref[...])
    h = jax.nn.relu(jnp.dot(h, wh1_ref[...],
                            preferred_element_type=jnp.float32) + bh1_ref[...])
    o_ref[...] = jnp.dot(h, wh2_ref[...],
                         preferred_element_type=jnp.float32) + bh2_ref[...]


def _tc_head(part, m, b, wh1, bh1, wh2, bh2):
    return pl.pallas_call(
        _head_body,
        grid=(_GRID,),
        in_specs=[pl.BlockSpec((1, _BLK, D), lambda i: (0, i, 0)),
                  pl.BlockSpec((1, _BLK, D), lambda i: (1, i, 0)),
                  pl.BlockSpec((_BLK, D), lambda i: (i, 0)),
                  pl.BlockSpec((1, D), lambda i: (0, 0)),
                  pl.BlockSpec((D, D), lambda i: (0, 0)),
                  pl.BlockSpec((1, D), lambda i: (0, 0)),
                  pl.BlockSpec((D, D), lambda i: (0, 0)),
                  pl.BlockSpec((1, D), lambda i: (0, 0))],
        out_specs=pl.BlockSpec((_BLK, D), lambda i: (i, 0)),
        out_shape=jax.ShapeDtypeStruct((N, D), jnp.float32),
    )(part, part, m, b, wh1, bh1, wh2, bh2)


def kernel(x, edge_index, W1, b1, W2, b2, Wh1, bh1, Wh2, bh2):
    src_p, dst_p = _tc_prep(edge_index.reshape(2, E // GROUP, GROUP))

    b1r = b1.reshape(1, D)
    b2r = b2.reshape(1, D)
    bh1r = bh1.reshape(1, D)
    bh2r = bh2.reshape(1, D)

    m1 = _tc_matmul(x, W1)
    p1 = _sc_edge_agg(m1, src_p, dst_p)
    m2 = _tc_combine_matmul(p1, m1, b1r, W2)
    p2 = _sc_edge_agg(m2, src_p, dst_p)
    return _tc_head(p2, m2, b2r, Wh1, bh1r, Wh2, bh2r)
